# baseline (speedup 1.0000x reference)
import jax
import jax.numpy as jnp
from jax import lax
from jax.experimental import pallas as pl
from jax.experimental.pallas import tpu as pltpu
from jax.experimental.pallas import tpu_sc as plsc

N = 10000
E = 320000
D = 128
H = 64
CP = 16
DR = 640

NC = 2
NS = 16
NW = NC * NS
CH = 128
CPW = -(-E // (CH * NW))
EPAD = NW * CPW * CH
NA = N + 16
RPT = 640
RPT_LAST = N - (NS - 1) * RPT
BLK = 1000
GRID = N // BLK


def _make_sc_aggregate(width, with_deg):
  mesh = plsc.VectorSubcoreMesh(
      core_axis_name="c", subcore_axis_name="s", num_cores=NC, num_subcores=NS)
  out_type = [jax.ShapeDtypeStruct((NC, N, width), jnp.float32)]
  scratch = [
      pltpu.VMEM((CPW, CH), jnp.int32),
      pltpu.VMEM((CPW, CH), jnp.int32),
      pltpu.VMEM((CH, width), jnp.float32),
      pltpu.VMEM((CH, width), jnp.float32),
      pltpu.VMEM((CH, width), jnp.float32),
      pltpu.VMEM((CH, width), jnp.float32),
      pltpu.VMEM_SHARED((NA, width), jnp.float32),
      pltpu.SemaphoreType.DMA,
      pltpu.SemaphoreType.DMA,
      pltpu.SemaphoreType.DMA,
      pltpu.SemaphoreType.DMA,
      pltpu.SemaphoreType.DMA,
      pltpu.SemaphoreType.DMA,
      pltpu.SemaphoreType.DMA,
      pltpu.SemaphoreType.DMA,
  ]
  if with_deg:
    out_type.append(jax.ShapeDtypeStruct((NC, DR, CP), jnp.float32))
    scratch += [
        pltpu.VMEM((DR, CP), jnp.float32),
        pltpu.VMEM_SHARED((DR, CP), jnp.float32),
        pltpu.VMEM((DR // 128, 128), jnp.int32),
    ]

  def _deg_accum(deg2d, didx, j):
    one = jnp.ones((16,), jnp.float32)
    for k in range(CH // 16):
      dstv = didx[j, pl.ds(k * 16, 16)]
      plsc.addupdate_scatter(
          deg2d, [lax.shift_right_logical(dstv, 4),
                  jnp.bitwise_and(dstv, 15)], one)

  def _body(y_hbm, src_hbm, dst_hbm, zacc_hbm, zdeg_hbm, iota_hbm,
            acc_out, deg_out, sidx, didx, bufs, acc,
            gsems, ssems, deg2d, dega, iota_v):
    c = lax.axis_index("c")
    s = lax.axis_index("s")
    wid = c * NS + s
    row0 = s * RPT

    def _init(nrows):
      pltpu.sync_copy(zacc_hbm.at[pl.ds(row0, nrows)], acc.at[pl.ds(row0, nrows)])

    pl.when(s < NS - 1)(lambda: _init(RPT))
    pl.when(s == NS - 1)(lambda: _init(RPT_LAST))
    pltpu.sync_copy(src_hbm.at[wid], sidx)
    pltpu.sync_copy(dst_hbm.at[wid], didx)
    if with_deg:
      pltpu.sync_copy(zdeg_hbm.at[pl.ds(0, DR)], deg2d)
      drow = s * (DR // NS)
      pltpu.sync_copy(zdeg_hbm.at[pl.ds(drow, DR // NS)],
                      dega.at[pl.ds(drow, DR // NS)])
      pltpu.sync_copy(iota_hbm, iota_v)
    plsc.subcore_barrier()

    def _gather(chunk, b):
      pltpu.async_copy(y_hbm.at[sidx.at[chunk]], bufs[b], gsems[b])

    def _wait_gather(chunk, b):
      pltpu.make_async_copy(y_hbm.at[sidx.at[chunk]], bufs[b], gsems[b]).wait()

    def _scatter(chunk, b):
      pltpu.async_copy(bufs[b], acc.at[didx.at[chunk]], sem=ssems[b], add=True)

    def _wait_scatter(chunk, b):
      pltpu.make_async_copy(
          bufs[b], acc.at[didx.at[chunk]], ssems[b]).wait()

    _gather(0, 0)
    _gather(1, 1)

    @pl.loop(0, CPW - 3, step=4)
    def _loop(j):
      pl.when(j > 0)(lambda: _wait_scatter(j - 2, 2))
      _gather(j + 2, 2)
      pl.when(j > 0)(lambda: _wait_scatter(j - 1, 3))
      _gather(j + 3, 3)
      _wait_gather(j, 0)
      _scatter(j, 0)
      if with_deg:
        _deg_accum(deg2d, didx, j)
      _wait_gather(j + 1, 1)
      _scatter(j + 1, 1)
      if with_deg:
        _deg_accum(deg2d, didx, j + 1)
      _wait_scatter(j, 0)
      _gather(j + 4, 0)
      _wait_scatter(j + 1, 1)
      _gather(j + 5, 1)
      _wait_gather(j + 2, 2)
      _scatter(j + 2, 2)
      if with_deg:
        _deg_accum(deg2d, didx, j + 2)
      _wait_gather(j + 3, 3)
      _scatter(j + 3, 3)
      if with_deg:
        _deg_accum(deg2d, didx, j + 3)

    _wait_scatter(CPW - 5, 2)
    _gather(CPW - 1, 2)
    _wait_scatter(CPW - 4, 3)
    _wait_gather(CPW - 3, 0)
    _scatter(CPW - 3, 0)
    if with_deg:
      _deg_accum(deg2d, didx, CPW - 3)
    _wait_gather(CPW - 2, 1)
    _scatter(CPW - 2, 1)
    if with_deg:
      _deg_accum(deg2d, didx, CPW - 2)
    _wait_gather(CPW - 1, 2)
    _scatter(CPW - 1, 2)
    if with_deg:
      _deg_accum(deg2d, didx, CPW - 1)
    _wait_scatter(CPW - 3, 0)
    _wait_scatter(CPW - 2, 1)
    _wait_scatter(CPW - 1, 2)
    if with_deg:
      for b in range(DR // 128):
        pltpu.sync_copy(deg2d.at[pl.ds(b * 128, 128)],
                        dega.at[iota_v.at[b]], add=True)

    plsc.subcore_barrier()

    def _copy_out(nrows):
      pltpu.sync_copy(acc.at[pl.ds(row0, nrows)],
                      acc_out.at[c, pl.ds(row0, nrows)])

    pl.when(s < NS - 1)(lambda: _copy_out(RPT))
    pl.when(s == NS - 1)(lambda: _copy_out(RPT_LAST))
    if with_deg:
      drow = s * (DR // NS)
      pltpu.sync_copy(dega.at[pl.ds(drow, DR // NS)],
                      deg_out.at[c, pl.ds(drow, DR // NS)])

  if with_deg:
    def fn(y_hbm, src_hbm, dst_hbm, zacc_hbm, zdeg_hbm, iota_hbm,
           acc_out, deg_out, sidx, didx, b0, b1, b2, b3, acc,
           g0, g1, g2, g3, s0, s1, s2, s3, deg2d, dega, iota_v):
      _body(y_hbm, src_hbm, dst_hbm, zacc_hbm, zdeg_hbm, iota_hbm,
            acc_out, deg_out, sidx, didx, (b0, b1, b2, b3), acc,
            (g0, g1, g2, g3), (s0, s1, s2, s3), deg2d, dega, iota_v)
    out = tuple(out_type)
  else:
    def fn(y_hbm, src_hbm, dst_hbm, zacc_hbm,
           acc_out, sidx, didx, b0, b1, b2, b3, acc,
           g0, g1, g2, g3, s0, s1, s2, s3):
      _body(y_hbm, src_hbm, dst_hbm, zacc_hbm, None, None,
            acc_out, None, sidx, didx, (b0, b1, b2, b3), acc,
            (g0, g1, g2, g3), (s0, s1, s2, s3), None, None, None)
    out = out_type[0]

  return pl.kernel(
      fn,
      out_type=out,
      mesh=mesh,
      scratch_types=scratch,
      compiler_params=pltpu.CompilerParams(
          use_tc_tiling_on_sc=False, needs_layout_passes=False),
  )


_SC_CACHE = {}


def _sc_aggregate(width, with_deg):
  key = (width, with_deg)
  if key not in _SC_CACHE:
    _SC_CACHE[key] = _make_sc_aggregate(width, with_deg)
  return _SC_CACHE[key]


def _tc1_body(x_ref, w_ref, b_ref, y_ref, r_ref):
  yr = jnp.dot(x_ref[...], w_ref[...], preferred_element_type=jnp.float32)
  y_ref[...] = yr[:, :H]
  r_ref[...] = yr[:, H:] + b_ref[...]


_tc1 = pl.pallas_call(
    _tc1_body,
    grid=(GRID,),
    in_specs=[
        pl.BlockSpec((BLK, D), lambda i: (i, 0)),
        pl.BlockSpec((D, 2 * H), lambda i: (0, 0)),
        pl.BlockSpec((1, H), lambda i: (0, 0)),
    ],
    out_specs=[
        pl.BlockSpec((BLK, H), lambda i: (i, 0)),
        pl.BlockSpec((BLK, H), lambda i: (i, 0)),
    ],
    out_shape=[
        jax.ShapeDtypeStruct((N, H), jnp.float32),
        jax.ShapeDtypeStruct((N, H), jnp.float32),
    ],
)


def _tc2_body(accp_ref, degp_ref, r1_ref, w_ref, b_ref, y2_ref, r2_ref):
  acc = accp_ref[0] + accp_ref[1]
  deg = degp_ref[0] + degp_ref[1]
  h = jnp.maximum(acc / jnp.maximum(deg, 1.0) + r1_ref[...], 0.0)
  yr = jnp.dot(h, w_ref[...], preferred_element_type=jnp.float32)
  y2_ref[...] = yr[:, :CP]
  r2_ref[...] = yr[:, CP:] + b_ref[...]


_tc2 = pl.pallas_call(
    _tc2_body,
    grid=(GRID,),
    in_specs=[
        pl.BlockSpec((NC, BLK, H), lambda i: (0, i, 0)),
        pl.BlockSpec((NC, BLK, 1), lambda i: (0, i, 0)),
        pl.BlockSpec((BLK, H), lambda i: (i, 0)),
        pl.BlockSpec((H, 2 * CP), lambda i: (0, 0)),
        pl.BlockSpec((1, CP), lambda i: (0, 0)),
    ],
    out_specs=[
        pl.BlockSpec((BLK, CP), lambda i: (i, 0)),
        pl.BlockSpec((BLK, CP), lambda i: (i, 0)),
    ],
    out_shape=[
        jax.ShapeDtypeStruct((N, CP), jnp.float32),
        jax.ShapeDtypeStruct((N, CP), jnp.float32),
    ],
)


def _tc3_body(accp_ref, degp_ref, r2_ref, out_ref):
  acc = accp_ref[0] + accp_ref[1]
  deg = degp_ref[0] + degp_ref[1]
  o = acc / jnp.maximum(deg, 1.0) + r2_ref[...]
  valid = lax.broadcasted_iota(jnp.int32, o.shape, 1) < 8
  om = jnp.where(valid, o, -jnp.inf)
  m = jnp.max(om, axis=1, keepdims=True)
  e = jnp.where(valid, jnp.exp(o - m), 0.0)
  lse = jnp.log(jnp.sum(e, axis=1, keepdims=True)) + m
  out_ref[...] = (o - lse)[:, :8]


_tc3 = pl.pallas_call(
    _tc3_body,
    grid=(GRID,),
    in_specs=[
        pl.BlockSpec((NC, BLK, CP), lambda i: (0, i, 0)),
        pl.BlockSpec((NC, BLK, 1), lambda i: (0, i, 0)),
        pl.BlockSpec((BLK, CP), lambda i: (i, 0)),
    ],
    out_specs=pl.BlockSpec((BLK, 8), lambda i: (i, 0)),
    out_shape=jax.ShapeDtypeStruct((N, 8), jnp.float32),
)


def kernel(x, edge_index, Wl1, Wr1, b1, Wl2, Wr2, b2):
  pad = EPAD - E
  src2d = jnp.concatenate(
      [edge_index[0], jnp.zeros((pad,), jnp.int32)]).reshape(NW, CPW, CH)
  dst2d = jnp.concatenate(
      [edge_index[1], jnp.full((pad,), N, jnp.int32)]).reshape(NW, CPW, CH)
  zH = jnp.zeros((N, H), jnp.float32)
  zC = jnp.zeros((N, CP), jnp.float32)
  iota = jnp.arange(DR, dtype=jnp.int32).reshape(DR // 128, 128)

  y1, r1 = _tc1(x, jnp.concatenate([Wl1, Wr1], axis=1), b1.reshape(1, H))
  accp, degg = _sc_aggregate(H, True)(y1, src2d, dst2d, zH, zC, iota)
  degp = degg.reshape(NC, DR * CP)[:, :N].reshape(NC, N, 1)

  W2 = jnp.concatenate(
      [jnp.pad(Wl2, ((0, 0), (0, CP - 8))),
       jnp.pad(Wr2, ((0, 0), (0, CP - 8)))], axis=1)
  b2p = jnp.pad(b2, (0, CP - 8)).reshape(1, CP)
  y2, r2 = _tc2(accp, degp, r1, W2, b2p)

  acc2p = _sc_aggregate(CP, False)(y2, src2d, dst2d, zC)
  return _tc3(acc2p, degp, r2)


# speedup vs baseline: 1.5239x; 1.5239x over previous
import jax
import jax.numpy as jnp
from jax import lax
from jax.experimental import pallas as pl
from jax.experimental.pallas import tpu as pltpu
from jax.experimental.pallas import tpu_sc as plsc

N = 10000
E = 320000
D = 128
H = 64
CP = 16
DR = 640

NC = 2
NS = 16
NW = NC * NS
CH = 80
CPW = E // (CH * NW)
NA = N
RPT = 640
RPT_LAST = N - (NS - 1) * RPT
BLK = 1000
GRID = N // BLK


def _make_sc_aggregate(width, with_deg):
  mesh = plsc.VectorSubcoreMesh(
      core_axis_name="c", subcore_axis_name="s", num_cores=NC, num_subcores=NS)
  out_type = [jax.ShapeDtypeStruct((NC, N, width), jnp.float32)]
  scratch = [
      pltpu.VMEM((CPW, CH), jnp.int32),
      pltpu.VMEM((CPW, CH), jnp.int32),
      pltpu.VMEM((CH, width), jnp.float32),
      pltpu.VMEM((CH, width), jnp.float32),
      pltpu.VMEM((CH, width), jnp.float32),
      pltpu.VMEM((CH, width), jnp.float32),
      pltpu.VMEM_SHARED((NA, width), jnp.float32),
      pltpu.SemaphoreType.DMA,
      pltpu.SemaphoreType.DMA,
      pltpu.SemaphoreType.DMA,
      pltpu.SemaphoreType.DMA,
      pltpu.SemaphoreType.DMA,
      pltpu.SemaphoreType.DMA,
      pltpu.SemaphoreType.DMA,
      pltpu.SemaphoreType.DMA,
  ]
  if with_deg:
    out_type.append(jax.ShapeDtypeStruct((NC, DR, CP), jnp.float32))
    scratch += [
        pltpu.VMEM((DR, CP), jnp.float32),
        pltpu.VMEM_SHARED((DR, CP), jnp.float32),
        pltpu.VMEM((DR // 128, 128), jnp.int32),
    ]

  def _deg_accum(deg2d, didx, j):
    one = jnp.ones((16,), jnp.float32)
    for k in range(CH // 16):
      dstv = didx[j, pl.ds(k * 16, 16)]
      plsc.addupdate_scatter(
          deg2d, [lax.shift_right_logical(dstv, 4),
                  jnp.bitwise_and(dstv, 15)], one)

  def _body(y_hbm, src_hbm, dst_hbm, zacc_hbm, zdeg_hbm, iota_hbm,
            acc_out, deg_out, sidx, didx, bufs, acc,
            gsems, ssems, deg2d, dega, iota_v):
    c = lax.axis_index("c")
    s = lax.axis_index("s")
    wid = c * NS + s
    row0 = s * RPT

    def _init(nrows):
      pltpu.sync_copy(zacc_hbm.at[pl.ds(row0, nrows)], acc.at[pl.ds(row0, nrows)])

    pl.when(s < NS - 1)(lambda: _init(RPT))
    pl.when(s == NS - 1)(lambda: _init(RPT_LAST))
    pltpu.sync_copy(src_hbm.at[wid], sidx)
    pltpu.sync_copy(dst_hbm.at[wid], didx)
    if with_deg:
      pltpu.sync_copy(zdeg_hbm.at[pl.ds(0, DR)], deg2d)
      drow = s * (DR // NS)
      pltpu.sync_copy(zdeg_hbm.at[pl.ds(drow, DR // NS)],
                      dega.at[pl.ds(drow, DR // NS)])
      pltpu.sync_copy(iota_hbm, iota_v)
    plsc.subcore_barrier()

    def _gather(chunk, b):
      pltpu.async_copy(y_hbm.at[sidx.at[chunk]], bufs[b], gsems[b])

    def _wait_gather(chunk, b):
      pltpu.make_async_copy(y_hbm.at[sidx.at[chunk]], bufs[b], gsems[b]).wait()

    def _scatter(chunk, b):
      pltpu.async_copy(bufs[b], acc.at[didx.at[chunk]], sem=ssems[b], add=True)

    def _wait_scatter(chunk, b):
      pltpu.make_async_copy(
          bufs[b], acc.at[didx.at[chunk]], ssems[b]).wait()

    _gather(0, 0)
    _gather(1, 1)

    @pl.loop(0, CPW - 1, step=4)
    def _loop(j):
      pl.when(j > 0)(lambda: _wait_scatter(j - 2, 2))
      _gather(j + 2, 2)
      pl.when(j > 0)(lambda: _wait_scatter(j - 1, 3))
      _gather(j + 3, 3)
      _wait_gather(j, 0)
      _scatter(j, 0)
      if with_deg:
        _deg_accum(deg2d, didx, j)
      _wait_gather(j + 1, 1)
      _scatter(j + 1, 1)
      if with_deg:
        _deg_accum(deg2d, didx, j + 1)
      _wait_scatter(j, 0)
      _gather(j + 4, 0)
      pl.when(j < CPW - 5)(lambda: (_wait_scatter(j + 1, 1),
                                    _gather(j + 5, 1), None)[-1])
      _wait_gather(j + 2, 2)
      _scatter(j + 2, 2)
      if with_deg:
        _deg_accum(deg2d, didx, j + 2)
      _wait_gather(j + 3, 3)
      _scatter(j + 3, 3)
      if with_deg:
        _deg_accum(deg2d, didx, j + 3)

    _wait_gather(CPW - 1, 0)
    _scatter(CPW - 1, 0)
    if with_deg:
      _deg_accum(deg2d, didx, CPW - 1)
    _wait_scatter(CPW - 4, 1)
    _wait_scatter(CPW - 3, 2)
    _wait_scatter(CPW - 2, 3)
    _wait_scatter(CPW - 1, 0)
    if with_deg:
      for b in range(DR // 128):
        pltpu.sync_copy(deg2d.at[pl.ds(b * 128, 128)],
                        dega.at[iota_v.at[b]], add=True)

    plsc.subcore_barrier()

    def _copy_out(nrows):
      pltpu.sync_copy(acc.at[pl.ds(row0, nrows)],
                      acc_out.at[c, pl.ds(row0, nrows)])

    pl.when(s < NS - 1)(lambda: _copy_out(RPT))
    pl.when(s == NS - 1)(lambda: _copy_out(RPT_LAST))
    if with_deg:
      drow = s * (DR // NS)
      pltpu.sync_copy(dega.at[pl.ds(drow, DR // NS)],
                      deg_out.at[c, pl.ds(drow, DR // NS)])

  if with_deg:
    def fn(y_hbm, src_hbm, dst_hbm, zacc_hbm, zdeg_hbm, iota_hbm,
           acc_out, deg_out, sidx, didx, b0, b1, b2, b3, acc,
           g0, g1, g2, g3, s0, s1, s2, s3, deg2d, dega, iota_v):
      _body(y_hbm, src_hbm, dst_hbm, zacc_hbm, zdeg_hbm, iota_hbm,
            acc_out, deg_out, sidx, didx, (b0, b1, b2, b3), acc,
            (g0, g1, g2, g3), (s0, s1, s2, s3), deg2d, dega, iota_v)
    out = tuple(out_type)
  else:
    def fn(y_hbm, src_hbm, dst_hbm, zacc_hbm,
           acc_out, sidx, didx, b0, b1, b2, b3, acc,
           g0, g1, g2, g3, s0, s1, s2, s3):
      _body(y_hbm, src_hbm, dst_hbm, zacc_hbm, None, None,
            acc_out, None, sidx, didx, (b0, b1, b2, b3), acc,
            (g0, g1, g2, g3), (s0, s1, s2, s3), None, None, None)
    out = out_type[0]

  return pl.kernel(
      fn,
      out_type=out,
      mesh=mesh,
      scratch_types=scratch,
      compiler_params=pltpu.CompilerParams(
          use_tc_tiling_on_sc=False, needs_layout_passes=False),
  )


_SC_CACHE = {}


def _sc_aggregate(width, with_deg):
  key = (width, with_deg)
  if key not in _SC_CACHE:
    _SC_CACHE[key] = _make_sc_aggregate(width, with_deg)
  return _SC_CACHE[key]


def _tc1_body(x_ref, w_ref, b_ref, y_ref, r_ref):
  yr = jnp.dot(x_ref[...], w_ref[...], preferred_element_type=jnp.float32)
  y_ref[...] = yr[:, :H]
  r_ref[...] = yr[:, H:] + b_ref[...]


_tc1 = pl.pallas_call(
    _tc1_body,
    grid=(GRID,),
    in_specs=[
        pl.BlockSpec((BLK, D), lambda i: (i, 0)),
        pl.BlockSpec((D, 2 * H), lambda i: (0, 0)),
        pl.BlockSpec((1, H), lambda i: (0, 0)),
    ],
    out_specs=[
        pl.BlockSpec((BLK, H), lambda i: (i, 0)),
        pl.BlockSpec((BLK, H), lambda i: (i, 0)),
    ],
    out_shape=[
        jax.ShapeDtypeStruct((N, H), jnp.float32),
        jax.ShapeDtypeStruct((N, H), jnp.float32),
    ],
)


def _tc2_body(accp_ref, degp_ref, r1_ref, w_ref, b_ref, y2_ref, r2_ref):
  acc = accp_ref[0] + accp_ref[1]
  deg = degp_ref[0] + degp_ref[1]
  h = jnp.maximum(acc / jnp.maximum(deg, 1.0) + r1_ref[...], 0.0)
  yr = jnp.dot(h, w_ref[...], preferred_element_type=jnp.float32)
  y2_ref[...] = yr[:, :CP]
  r2_ref[...] = yr[:, CP:] + b_ref[...]


_tc2 = pl.pallas_call(
    _tc2_body,
    grid=(GRID,),
    in_specs=[
        pl.BlockSpec((NC, BLK, H), lambda i: (0, i, 0)),
        pl.BlockSpec((NC, BLK, 1), lambda i: (0, i, 0)),
        pl.BlockSpec((BLK, H), lambda i: (i, 0)),
        pl.BlockSpec((H, 2 * CP), lambda i: (0, 0)),
        pl.BlockSpec((1, CP), lambda i: (0, 0)),
    ],
    out_specs=[
        pl.BlockSpec((BLK, CP), lambda i: (i, 0)),
        pl.BlockSpec((BLK, CP), lambda i: (i, 0)),
    ],
    out_shape=[
        jax.ShapeDtypeStruct((N, CP), jnp.float32),
        jax.ShapeDtypeStruct((N, CP), jnp.float32),
    ],
)


def _tc3_body(accp_ref, degp_ref, r2_ref, out_ref):
  acc = accp_ref[0] + accp_ref[1]
  deg = degp_ref[0] + degp_ref[1]
  o = acc / jnp.maximum(deg, 1.0) + r2_ref[...]
  valid = lax.broadcasted_iota(jnp.int32, o.shape, 1) < 8
  om = jnp.where(valid, o, -jnp.inf)
  m = jnp.max(om, axis=1, keepdims=True)
  e = jnp.where(valid, jnp.exp(o - m), 0.0)
  lse = jnp.log(jnp.sum(e, axis=1, keepdims=True)) + m
  out_ref[...] = (o - lse)[:, :8]


_tc3 = pl.pallas_call(
    _tc3_body,
    grid=(GRID,),
    in_specs=[
        pl.BlockSpec((NC, BLK, CP), lambda i: (0, i, 0)),
        pl.BlockSpec((NC, BLK, 1), lambda i: (0, i, 0)),
        pl.BlockSpec((BLK, CP), lambda i: (i, 0)),
    ],
    out_specs=pl.BlockSpec((BLK, 8), lambda i: (i, 0)),
    out_shape=jax.ShapeDtypeStruct((N, 8), jnp.float32),
)


def kernel(x, edge_index, Wl1, Wr1, b1, Wl2, Wr2, b2):
  src2d = edge_index[0].reshape(NW, CPW, CH)
  dst2d = edge_index[1].reshape(NW, CPW, CH)
  zH = jnp.zeros((N, H), jnp.float32)
  zC = jnp.zeros((N, CP), jnp.float32)
  iota = jnp.arange(DR, dtype=jnp.int32).reshape(DR // 128, 128)

  y1, r1 = _tc1(x, jnp.concatenate([Wl1, Wr1], axis=1), b1.reshape(1, H))
  accp, degg = _sc_aggregate(H, True)(y1, src2d, dst2d, zH, zC, iota)
  degp = degg.reshape(NC, DR * CP)[:, :N].reshape(NC, N, 1)

  W2 = jnp.concatenate(
      [jnp.pad(Wl2, ((0, 0), (0, CP - 8))),
       jnp.pad(Wr2, ((0, 0), (0, CP - 8)))], axis=1)
  b2p = jnp.pad(b2, (0, CP - 8)).reshape(1, CP)
  y2, r2 = _tc2(accp, degp, r1, W2, b2p)

  acc2p = _sc_aggregate(CP, False)(y2, src2d, dst2d, zC)
  return _tc3(acc2p, degp, r2)


# speedup vs baseline: 1.5602x; 1.0238x over previous
import jax
import jax.numpy as jnp
from jax import lax
from jax.experimental import pallas as pl
from jax.experimental.pallas import tpu as pltpu
from jax.experimental.pallas import tpu_sc as plsc

N = 10000
E = 320000
D = 128
H = 64
CP = 16
DR = 640

NC = 2
NS = 16
NW = NC * NS
CH = 80
CPW = E // (CH * NW)
NA = N
RPT = 640
RPT_LAST = N - (NS - 1) * RPT
BLK = 1000
GRID = N // BLK


def _make_sc_aggregate(width, with_deg):
  mesh = plsc.VectorSubcoreMesh(
      core_axis_name="c", subcore_axis_name="s", num_cores=NC, num_subcores=NS)
  out_type = [jax.ShapeDtypeStruct((NC, N, width), jnp.float32)]
  scratch = [
      pltpu.VMEM((CPW, CH), jnp.int32),
      pltpu.VMEM((CPW, CH), jnp.int32),
      pltpu.VMEM((CH, width), jnp.float32),
      pltpu.VMEM((CH, width), jnp.float32),
      pltpu.VMEM((CH, width), jnp.float32),
      pltpu.VMEM((CH, width), jnp.float32),
      pltpu.VMEM_SHARED((NA, width), jnp.float32),
      pltpu.SemaphoreType.DMA,
      pltpu.SemaphoreType.DMA,
      pltpu.SemaphoreType.DMA,
      pltpu.SemaphoreType.DMA,
      pltpu.SemaphoreType.DMA,
      pltpu.SemaphoreType.DMA,
      pltpu.SemaphoreType.DMA,
      pltpu.SemaphoreType.DMA,
      pltpu.SemaphoreType.DMA,
  ]
  if with_deg:
    out_type.append(jax.ShapeDtypeStruct((NC, DR, CP), jnp.float32))
    scratch += [
        pltpu.VMEM((DR, CP), jnp.float32),
        pltpu.VMEM_SHARED((DR, CP), jnp.float32),
        pltpu.VMEM((DR // 128, 128), jnp.int32),
    ]

  def _deg_accum(deg2d, didx, j):
    one = jnp.ones((16,), jnp.float32)
    for k in range(CH // 16):
      dstv = didx[j, pl.ds(k * 16, 16)]
      plsc.addupdate_scatter(
          deg2d, [lax.shift_right_logical(dstv, 4),
                  jnp.bitwise_and(dstv, 15)], one)

  def _body(y_hbm, src_hbm, dst_hbm, zacc_hbm, zdeg_hbm, iota_hbm,
            acc_out, deg_out, sidx, didx, bufs, acc,
            gsems, ssems, zsem, deg2d, dega, iota_v):
    c = lax.axis_index("c")
    s = lax.axis_index("s")
    wid = c * NS + s
    row0 = s * RPT

    def _gather(chunk, b):
      pltpu.async_copy(y_hbm.at[sidx.at[chunk]], bufs[b], gsems[b])

    def _wait_gather(chunk, b):
      pltpu.make_async_copy(y_hbm.at[sidx.at[chunk]], bufs[b], gsems[b]).wait()

    def _scatter(chunk, b):
      pltpu.async_copy(bufs[b], acc.at[didx.at[chunk]], sem=ssems[b], add=True)

    def _wait_scatter(chunk, b):
      pltpu.make_async_copy(
          bufs[b], acc.at[didx.at[chunk]], ssems[b]).wait()

    pltpu.sync_copy(src_hbm.at[wid], sidx)
    pltpu.sync_copy(dst_hbm.at[wid], didx)
    _gather(0, 0)
    _gather(1, 1)
    inits = []

    def _init(nrows):
      pltpu.async_copy(zacc_hbm.at[pl.ds(row0, nrows)],
                       acc.at[pl.ds(row0, nrows)], zsem)

    pl.when(s < NS - 1)(lambda: _init(RPT))
    pl.when(s == NS - 1)(lambda: _init(RPT_LAST))
    if with_deg:
      drow = s * (DR // NS)
      inits.append(pltpu.async_copy(zdeg_hbm.at[pl.ds(0, DR)], deg2d, zsem))
      inits.append(pltpu.async_copy(zdeg_hbm.at[pl.ds(drow, DR // NS)],
                                    dega.at[pl.ds(drow, DR // NS)], zsem))
      inits.append(pltpu.async_copy(iota_hbm, iota_v, zsem))
    for d in inits:
      d.wait()
    pl.when(s < NS - 1)(
        lambda: pltpu.make_async_copy(zacc_hbm.at[pl.ds(row0, RPT)],
                                      acc.at[pl.ds(row0, RPT)], zsem).wait())
    pl.when(s == NS - 1)(
        lambda: pltpu.make_async_copy(zacc_hbm.at[pl.ds(row0, RPT_LAST)],
                                      acc.at[pl.ds(row0, RPT_LAST)],
                                      zsem).wait())
    plsc.subcore_barrier()

    @pl.loop(0, CPW - 1, step=4)
    def _loop(j):
      pl.when(j > 0)(lambda: _wait_scatter(j - 2, 2))
      _gather(j + 2, 2)
      pl.when(j > 0)(lambda: _wait_scatter(j - 1, 3))
      _gather(j + 3, 3)
      _wait_gather(j, 0)
      _scatter(j, 0)
      if with_deg:
        _deg_accum(deg2d, didx, j)
      _wait_gather(j + 1, 1)
      _scatter(j + 1, 1)
      if with_deg:
        _deg_accum(deg2d, didx, j + 1)
      _wait_scatter(j, 0)
      _gather(j + 4, 0)
      pl.when(j < CPW - 5)(lambda: (_wait_scatter(j + 1, 1),
                                    _gather(j + 5, 1), None)[-1])
      _wait_gather(j + 2, 2)
      _scatter(j + 2, 2)
      if with_deg:
        _deg_accum(deg2d, didx, j + 2)
      _wait_gather(j + 3, 3)
      _scatter(j + 3, 3)
      if with_deg:
        _deg_accum(deg2d, didx, j + 3)

    _wait_gather(CPW - 1, 0)
    _scatter(CPW - 1, 0)
    if with_deg:
      _deg_accum(deg2d, didx, CPW - 1)
    _wait_scatter(CPW - 4, 1)
    _wait_scatter(CPW - 3, 2)
    _wait_scatter(CPW - 2, 3)
    _wait_scatter(CPW - 1, 0)
    if with_deg:
      for b in range(DR // 128):
        pltpu.sync_copy(deg2d.at[pl.ds(b * 128, 128)],
                        dega.at[iota_v.at[b]], add=True)

    plsc.subcore_barrier()

    def _copy_out(nrows):
      pltpu.sync_copy(acc.at[pl.ds(row0, nrows)],
                      acc_out.at[c, pl.ds(row0, nrows)])

    pl.when(s < NS - 1)(lambda: _copy_out(RPT))
    pl.when(s == NS - 1)(lambda: _copy_out(RPT_LAST))
    if with_deg:
      drow = s * (DR // NS)
      pltpu.sync_copy(dega.at[pl.ds(drow, DR // NS)],
                      deg_out.at[c, pl.ds(drow, DR // NS)])

  if with_deg:
    def fn(y_hbm, src_hbm, dst_hbm, zacc_hbm, zdeg_hbm, iota_hbm,
           acc_out, deg_out, sidx, didx, b0, b1, b2, b3, acc,
           g0, g1, g2, g3, s0, s1, s2, s3, zsem, deg2d, dega, iota_v):
      _body(y_hbm, src_hbm, dst_hbm, zacc_hbm, zdeg_hbm, iota_hbm,
            acc_out, deg_out, sidx, didx, (b0, b1, b2, b3), acc,
            (g0, g1, g2, g3), (s0, s1, s2, s3), zsem, deg2d, dega, iota_v)
    out = tuple(out_type)
  else:
    def fn(y_hbm, src_hbm, dst_hbm, zacc_hbm,
           acc_out, sidx, didx, b0, b1, b2, b3, acc,
           g0, g1, g2, g3, s0, s1, s2, s3, zsem):
      _body(y_hbm, src_hbm, dst_hbm, zacc_hbm, None, None,
            acc_out, None, sidx, didx, (b0, b1, b2, b3), acc,
            (g0, g1, g2, g3), (s0, s1, s2, s3), zsem, None, None, None)
    out = out_type[0]

  return pl.kernel(
      fn,
      out_type=out,
      mesh=mesh,
      scratch_types=scratch,
      compiler_params=pltpu.CompilerParams(
          use_tc_tiling_on_sc=False, needs_layout_passes=False),
  )


_SC_CACHE = {}


def _sc_aggregate(width, with_deg):
  key = (width, with_deg)
  if key not in _SC_CACHE:
    _SC_CACHE[key] = _make_sc_aggregate(width, with_deg)
  return _SC_CACHE[key]


def _tc1_body(x_ref, w_ref, b_ref, y_ref, r_ref):
  yr = jnp.dot(x_ref[...], w_ref[...], preferred_element_type=jnp.float32)
  y_ref[...] = yr[:, :H]
  r_ref[...] = yr[:, H:] + b_ref[...]


_tc1 = pl.pallas_call(
    _tc1_body,
    grid=(GRID,),
    in_specs=[
        pl.BlockSpec((BLK, D), lambda i: (i, 0)),
        pl.BlockSpec((D, 2 * H), lambda i: (0, 0)),
        pl.BlockSpec((1, H), lambda i: (0, 0)),
    ],
    out_specs=[
        pl.BlockSpec((BLK, H), lambda i: (i, 0)),
        pl.BlockSpec((BLK, H), lambda i: (i, 0)),
    ],
    out_shape=[
        jax.ShapeDtypeStruct((N, H), jnp.float32),
        jax.ShapeDtypeStruct((N, H), jnp.float32),
    ],
)


def _tc2_body(accp_ref, degp_ref, r1_ref, w_ref, b_ref, y2_ref, r2_ref):
  acc = accp_ref[0] + accp_ref[1]
  deg = degp_ref[0] + degp_ref[1]
  h = jnp.maximum(acc / jnp.maximum(deg, 1.0) + r1_ref[...], 0.0)
  yr = jnp.dot(h, w_ref[...], preferred_element_type=jnp.float32)
  y2_ref[...] = yr[:, :CP]
  r2_ref[...] = yr[:, CP:] + b_ref[...]


_tc2 = pl.pallas_call(
    _tc2_body,
    grid=(GRID,),
    in_specs=[
        pl.BlockSpec((NC, BLK, H), lambda i: (0, i, 0)),
        pl.BlockSpec((NC, BLK, 1), lambda i: (0, i, 0)),
        pl.BlockSpec((BLK, H), lambda i: (i, 0)),
        pl.BlockSpec((H, 2 * CP), lambda i: (0, 0)),
        pl.BlockSpec((1, CP), lambda i: (0, 0)),
    ],
    out_specs=[
        pl.BlockSpec((BLK, CP), lambda i: (i, 0)),
        pl.BlockSpec((BLK, CP), lambda i: (i, 0)),
    ],
    out_shape=[
        jax.ShapeDtypeStruct((N, CP), jnp.float32),
        jax.ShapeDtypeStruct((N, CP), jnp.float32),
    ],
)


def _tc3_body(accp_ref, degp_ref, r2_ref, out_ref):
  acc = accp_ref[0] + accp_ref[1]
  deg = degp_ref[0] + degp_ref[1]
  o = acc / jnp.maximum(deg, 1.0) + r2_ref[...]
  valid = lax.broadcasted_iota(jnp.int32, o.shape, 1) < 8
  om = jnp.where(valid, o, -jnp.inf)
  m = jnp.max(om, axis=1, keepdims=True)
  e = jnp.where(valid, jnp.exp(o - m), 0.0)
  lse = jnp.log(jnp.sum(e, axis=1, keepdims=True)) + m
  out_ref[...] = (o - lse)[:, :8]


_tc3 = pl.pallas_call(
    _tc3_body,
    grid=(GRID,),
    in_specs=[
        pl.BlockSpec((NC, BLK, CP), lambda i: (0, i, 0)),
        pl.BlockSpec((NC, BLK, 1), lambda i: (0, i, 0)),
        pl.BlockSpec((BLK, CP), lambda i: (i, 0)),
    ],
    out_specs=pl.BlockSpec((BLK, 8), lambda i: (i, 0)),
    out_shape=jax.ShapeDtypeStruct((N, 8), jnp.float32),
)


def kernel(x, edge_index, Wl1, Wr1, b1, Wl2, Wr2, b2):
  src2d = edge_index[0].reshape(NW, CPW, CH)
  dst2d = edge_index[1].reshape(NW, CPW, CH)
  zH = jnp.zeros((N, H), jnp.float32)
  zC = jnp.zeros((N, CP), jnp.float32)
  iota = jnp.arange(DR, dtype=jnp.int32).reshape(DR // 128, 128)

  y1, r1 = _tc1(x, jnp.concatenate([Wl1, Wr1], axis=1), b1.reshape(1, H))
  accp, degg = _sc_aggregate(H, True)(y1, src2d, dst2d, zH, zC, iota)
  degp = degg.reshape(NC, DR * CP)[:, :N].reshape(NC, N, 1)

  W2 = jnp.concatenate(
      [jnp.pad(Wl2, ((0, 0), (0, CP - 8))),
       jnp.pad(Wr2, ((0, 0), (0, CP - 8)))], axis=1)
  b2p = jnp.pad(b2, (0, CP - 8)).reshape(1, CP)
  y2, r2 = _tc2(accp, degp, r1, W2, b2p)

  acc2p = _sc_aggregate(CP, False)(y2, src2d, dst2d, zC)
  return _tc3(acc2p, degp, r2)


# speedup vs baseline: 1.6703x; 1.0706x over previous
import jax
import jax.numpy as jnp
from jax import lax
from jax.experimental import pallas as pl
from jax.experimental.pallas import tpu as pltpu
from jax.experimental.pallas import tpu_sc as plsc

N = 10000
E = 320000
D = 128
H = 64
CP = 16
DR = 640

NC = 2
NS = 16
NW = NC * NS
CH = 80
CPW = E // (CH * NW)
NA = N
RPT = 640
RPT_LAST = N - (NS - 1) * RPT
BLK = 2000
GRID = N // BLK


def _make_sc_aggregate(width, with_deg, out_width=None):
  out_width = out_width or width
  mesh = plsc.VectorSubcoreMesh(
      core_axis_name="c", subcore_axis_name="s", num_cores=NC, num_subcores=NS)
  out_type = [jax.ShapeDtypeStruct((NC, N, out_width), jnp.float32)]
  scratch = [
      pltpu.VMEM((CPW, CH), jnp.int32),
      pltpu.VMEM((CPW, CH), jnp.int32),
      pltpu.VMEM((CH, width), jnp.float32),
      pltpu.VMEM((CH, width), jnp.float32),
      pltpu.VMEM((CH, width), jnp.float32),
      pltpu.VMEM((CH, width), jnp.float32),
      pltpu.VMEM_SHARED((NA, width), jnp.float32),
      pltpu.SemaphoreType.DMA,
      pltpu.SemaphoreType.DMA,
      pltpu.SemaphoreType.DMA,
      pltpu.SemaphoreType.DMA,
      pltpu.SemaphoreType.DMA,
      pltpu.SemaphoreType.DMA,
      pltpu.SemaphoreType.DMA,
      pltpu.SemaphoreType.DMA,
      pltpu.SemaphoreType.DMA,
  ]
  if with_deg:
    out_type.append(jax.ShapeDtypeStruct((NC, DR, CP), jnp.float32))
    scratch += [
        pltpu.VMEM((DR, CP), jnp.float32),
        pltpu.VMEM_SHARED((DR, CP), jnp.float32),
        pltpu.VMEM((DR // 128, 128), jnp.int32),
    ]

  def _deg_accum(deg2d, didx, j):
    one = jnp.ones((16,), jnp.float32)
    for k in range(CH // 16):
      dstv = didx[j, pl.ds(k * 16, 16)]
      plsc.addupdate_scatter(
          deg2d, [lax.shift_right_logical(dstv, 4),
                  jnp.bitwise_and(dstv, 15)], one)

  def _body(y_hbm, src_hbm, dst_hbm, zacc_hbm, zdeg_hbm, iota_hbm,
            acc_out, deg_out, sidx, didx, bufs, acc,
            gsems, ssems, zsem, deg2d, dega, iota_v):
    c = lax.axis_index("c")
    s = lax.axis_index("s")
    wid = c * NS + s
    row0 = s * RPT

    def _gather(chunk, b):
      pltpu.async_copy(y_hbm.at[sidx.at[chunk]], bufs[b], gsems[b])

    def _wait_gather(chunk, b):
      pltpu.make_async_copy(y_hbm.at[sidx.at[chunk]], bufs[b], gsems[b]).wait()

    def _scatter(chunk, b):
      pltpu.async_copy(bufs[b], acc.at[didx.at[chunk]], sem=ssems[b], add=True)

    def _wait_scatter(chunk, b):
      pltpu.make_async_copy(
          bufs[b], acc.at[didx.at[chunk]], ssems[b]).wait()

    pltpu.sync_copy(src_hbm.at[wid], sidx)
    pltpu.sync_copy(dst_hbm.at[wid], didx)
    _gather(0, 0)
    _gather(1, 1)
    inits = []

    def _init(nrows):
      pltpu.async_copy(zacc_hbm.at[pl.ds(row0, nrows)],
                       acc.at[pl.ds(row0, nrows)], zsem)

    pl.when(s < NS - 1)(lambda: _init(RPT))
    pl.when(s == NS - 1)(lambda: _init(RPT_LAST))
    if with_deg:
      drow = s * (DR // NS)
      inits.append(pltpu.async_copy(zdeg_hbm.at[pl.ds(0, DR)], deg2d, zsem))
      inits.append(pltpu.async_copy(zdeg_hbm.at[pl.ds(drow, DR // NS)],
                                    dega.at[pl.ds(drow, DR // NS)], zsem))
      inits.append(pltpu.async_copy(iota_hbm, iota_v, zsem))
    for d in inits:
      d.wait()
    pl.when(s < NS - 1)(
        lambda: pltpu.make_async_copy(zacc_hbm.at[pl.ds(row0, RPT)],
                                      acc.at[pl.ds(row0, RPT)], zsem).wait())
    pl.when(s == NS - 1)(
        lambda: pltpu.make_async_copy(zacc_hbm.at[pl.ds(row0, RPT_LAST)],
                                      acc.at[pl.ds(row0, RPT_LAST)],
                                      zsem).wait())
    plsc.subcore_barrier()

    @pl.loop(0, CPW - 1, step=4)
    def _loop(j):
      pl.when(j > 0)(lambda: _wait_scatter(j - 2, 2))
      _gather(j + 2, 2)
      pl.when(j > 0)(lambda: _wait_scatter(j - 1, 3))
      _gather(j + 3, 3)
      _wait_gather(j, 0)
      _scatter(j, 0)
      if with_deg:
        _deg_accum(deg2d, didx, j)
      _wait_gather(j + 1, 1)
      _scatter(j + 1, 1)
      if with_deg:
        _deg_accum(deg2d, didx, j + 1)
      _wait_scatter(j, 0)
      _gather(j + 4, 0)
      pl.when(j < CPW - 5)(lambda: (_wait_scatter(j + 1, 1),
                                    _gather(j + 5, 1), None)[-1])
      _wait_gather(j + 2, 2)
      _scatter(j + 2, 2)
      if with_deg:
        _deg_accum(deg2d, didx, j + 2)
      _wait_gather(j + 3, 3)
      _scatter(j + 3, 3)
      if with_deg:
        _deg_accum(deg2d, didx, j + 3)

    _wait_gather(CPW - 1, 0)
    _scatter(CPW - 1, 0)
    if with_deg:
      _deg_accum(deg2d, didx, CPW - 1)
    _wait_scatter(CPW - 4, 1)
    _wait_scatter(CPW - 3, 2)
    _wait_scatter(CPW - 2, 3)
    _wait_scatter(CPW - 1, 0)
    if with_deg:
      for b in range(DR // 128):
        pltpu.sync_copy(deg2d.at[pl.ds(b * 128, 128)],
                        dega.at[iota_v.at[b]], add=True)

    plsc.subcore_barrier()

    def _copy_out(nrows):
      if out_width == width:
        pltpu.sync_copy(acc.at[pl.ds(row0, nrows)],
                        acc_out.at[c, pl.ds(row0, nrows)])
      else:
        pltpu.sync_copy(acc.at[pl.ds(row0, nrows)],
                        acc_out.at[c, pl.ds(row0, nrows), pl.ds(0, width)])

    pl.when(s < NS - 1)(lambda: _copy_out(RPT))
    pl.when(s == NS - 1)(lambda: _copy_out(RPT_LAST))
    if with_deg:
      drow = s * (DR // NS)
      pltpu.sync_copy(dega.at[pl.ds(drow, DR // NS)],
                      deg_out.at[c, pl.ds(drow, DR // NS)])

  if with_deg:
    def fn(y_hbm, src_hbm, dst_hbm, zacc_hbm, zdeg_hbm, iota_hbm,
           acc_out, deg_out, sidx, didx, b0, b1, b2, b3, acc,
           g0, g1, g2, g3, s0, s1, s2, s3, zsem, deg2d, dega, iota_v):
      _body(y_hbm, src_hbm, dst_hbm, zacc_hbm, zdeg_hbm, iota_hbm,
            acc_out, deg_out, sidx, didx, (b0, b1, b2, b3), acc,
            (g0, g1, g2, g3), (s0, s1, s2, s3), zsem, deg2d, dega, iota_v)
    out = tuple(out_type)
  else:
    def fn(y_hbm, src_hbm, dst_hbm, zacc_hbm,
           acc_out, sidx, didx, b0, b1, b2, b3, acc,
           g0, g1, g2, g3, s0, s1, s2, s3, zsem):
      _body(y_hbm, src_hbm, dst_hbm, zacc_hbm, None, None,
            acc_out, None, sidx, didx, (b0, b1, b2, b3), acc,
            (g0, g1, g2, g3), (s0, s1, s2, s3), zsem, None, None, None)
    out = out_type[0]

  return pl.kernel(
      fn,
      out_type=out,
      mesh=mesh,
      scratch_types=scratch,
      compiler_params=pltpu.CompilerParams(
          use_tc_tiling_on_sc=False, needs_layout_passes=False),
  )


_SC_CACHE = {}


def _sc_aggregate(width, with_deg, out_width=None):
  key = (width, with_deg, out_width)
  if key not in _SC_CACHE:
    _SC_CACHE[key] = _make_sc_aggregate(width, with_deg, out_width)
  return _SC_CACHE[key]


def _tc1_body(x_ref, w_ref, b_ref, y_ref, r_ref):
  yr = jnp.dot(x_ref[...], w_ref[...], preferred_element_type=jnp.float32)
  y_ref[...] = yr[:, :H]
  r_ref[...] = yr[:, H:] + b_ref[...]


_tc1 = pl.pallas_call(
    _tc1_body,
    grid=(GRID,),
    in_specs=[
        pl.BlockSpec((BLK, D), lambda i: (i, 0)),
        pl.BlockSpec((D, 2 * H), lambda i: (0, 0)),
        pl.BlockSpec((1, H), lambda i: (0, 0)),
    ],
    out_specs=[
        pl.BlockSpec((BLK, H), lambda i: (i, 0)),
        pl.BlockSpec((BLK, H), lambda i: (i, 0)),
    ],
    out_shape=[
        jax.ShapeDtypeStruct((N, H), jnp.float32),
        jax.ShapeDtypeStruct((N, H), jnp.float32),
    ],
)


def _tc2_body(accp_ref, degp_ref, r1_ref, w_ref, b_ref, y2_ref, r2_ref):
  acc = accp_ref[0, :, :H] + accp_ref[1, :, :H]
  deg = degp_ref[0] + degp_ref[1]
  h = jnp.maximum(acc / jnp.maximum(deg, 1.0) + r1_ref[...], 0.0)
  yr = jnp.dot(h, w_ref[...], preferred_element_type=jnp.float32)
  y2_ref[...] = yr[:, :CP]
  r2_ref[...] = yr[:, CP:] + b_ref[...]


_tc2 = pl.pallas_call(
    _tc2_body,
    grid=(GRID,),
    in_specs=[
        pl.BlockSpec((NC, BLK, 2 * H), lambda i: (0, i, 0)),
        pl.BlockSpec((NC, BLK, 1), lambda i: (0, i, 0)),
        pl.BlockSpec((BLK, H), lambda i: (i, 0)),
        pl.BlockSpec((H, 2 * CP), lambda i: (0, 0)),
        pl.BlockSpec((1, CP), lambda i: (0, 0)),
    ],
    out_specs=[
        pl.BlockSpec((BLK, CP), lambda i: (i, 0)),
        pl.BlockSpec((BLK, CP), lambda i: (i, 0)),
    ],
    out_shape=[
        jax.ShapeDtypeStruct((N, CP), jnp.float32),
        jax.ShapeDtypeStruct((N, CP), jnp.float32),
    ],
)


def _tc3_body(accp_ref, degp_ref, r2_ref, out_ref):
  acc = accp_ref[0] + accp_ref[1]
  deg = degp_ref[0] + degp_ref[1]
  o = acc / jnp.maximum(deg, 1.0) + r2_ref[...]
  valid = lax.broadcasted_iota(jnp.int32, o.shape, 1) < 8
  om = jnp.where(valid, o, -jnp.inf)
  m = jnp.max(om, axis=1, keepdims=True)
  e = jnp.where(valid, jnp.exp(o - m), 0.0)
  lse = jnp.log(jnp.sum(e, axis=1, keepdims=True)) + m
  out_ref[...] = (o - lse)[:, :8]


_tc3 = pl.pallas_call(
    _tc3_body,
    grid=(GRID,),
    in_specs=[
        pl.BlockSpec((NC, BLK, CP), lambda i: (0, i, 0)),
        pl.BlockSpec((NC, BLK, 1), lambda i: (0, i, 0)),
        pl.BlockSpec((BLK, CP), lambda i: (i, 0)),
    ],
    out_specs=pl.BlockSpec((BLK, 8), lambda i: (i, 0)),
    out_shape=jax.ShapeDtypeStruct((N, 8), jnp.float32),
)


def kernel(x, edge_index, Wl1, Wr1, b1, Wl2, Wr2, b2):
  src2d = edge_index[0].reshape(NW, CPW, CH)
  dst2d = edge_index[1].reshape(NW, CPW, CH)
  zH = jnp.zeros((N, H), jnp.float32)
  zC = jnp.zeros((N, CP), jnp.float32)
  iota = jnp.arange(DR, dtype=jnp.int32).reshape(DR // 128, 128)

  y1, r1 = _tc1(x, jnp.concatenate([Wl1, Wr1], axis=1), b1.reshape(1, H))
  accp, degg = _sc_aggregate(H, True, 2 * H)(y1, src2d, dst2d, zH, zC, iota)
  degp = degg.reshape(NC, DR * CP)[:, :N].reshape(NC, N, 1)

  W2 = jnp.concatenate(
      [jnp.pad(Wl2, ((0, 0), (0, CP - 8))),
       jnp.pad(Wr2, ((0, 0), (0, CP - 8)))], axis=1)
  b2p = jnp.pad(b2, (0, CP - 8)).reshape(1, CP)
  y2, r2 = _tc2(accp, degp, r1, W2, b2p)

  acc2p = _sc_aggregate(CP, False)(y2, src2d, dst2d, zC)
  return _tc3(acc2p, degp, r2)


# speedup vs baseline: 1.7075x; 1.0223x over previous
import jax
import jax.numpy as jnp
from jax import lax
from jax.experimental import pallas as pl
from jax.experimental.pallas import tpu as pltpu
from jax.experimental.pallas import tpu_sc as plsc

N = 10000
E = 320000
D = 128
H = 64
CP = 16
DR = 640

NC = 2
NS = 16
NW = NC * NS
CH = 80
CPW = E // (CH * NW)
NA = N
RPT = 640
RPT_LAST = N - (NS - 1) * RPT
BLK = 2000
GRID = N // BLK


def _make_sc_aggregate(width, with_deg, out_width=None):
  out_width = out_width or width
  mesh = plsc.VectorSubcoreMesh(
      core_axis_name="c", subcore_axis_name="s", num_cores=NC, num_subcores=NS)
  out_type = [jax.ShapeDtypeStruct((NC, N, out_width), jnp.float32)]
  scratch = [
      pltpu.VMEM((CPW, CH), jnp.int32),
      pltpu.VMEM((CPW, CH), jnp.int32),
      pltpu.VMEM((CH, width), jnp.float32),
      pltpu.VMEM((CH, width), jnp.float32),
      pltpu.VMEM((CH, width), jnp.float32),
      pltpu.VMEM((CH, width), jnp.float32),
      pltpu.VMEM_SHARED((NA, width), jnp.float32),
      pltpu.SemaphoreType.DMA,
      pltpu.SemaphoreType.DMA,
      pltpu.SemaphoreType.DMA,
      pltpu.SemaphoreType.DMA,
      pltpu.SemaphoreType.DMA,
      pltpu.SemaphoreType.DMA,
      pltpu.SemaphoreType.DMA,
      pltpu.SemaphoreType.DMA,
      pltpu.SemaphoreType.DMA,
  ]
  if with_deg:
    out_type.append(jax.ShapeDtypeStruct((NC, DR, CP), jnp.float32))
    scratch += [
        pltpu.VMEM((DR, CP), jnp.float32),
        pltpu.VMEM_SHARED((DR, CP), jnp.float32),
        pltpu.VMEM((DR // 128, 128), jnp.int32),
    ]

  def _deg_accum(deg2d, didx, j):
    one = jnp.ones((16,), jnp.float32)
    for k in range(CH // 16):
      dstv = didx[j, pl.ds(k * 16, 16)]
      plsc.addupdate_scatter(
          deg2d, [lax.shift_right_logical(dstv, 4),
                  jnp.bitwise_and(dstv, 15)], one)

  def _body(y_hbm, src_hbm, dst_hbm, zacc_hbm, zdeg_hbm, iota_hbm,
            acc_out, deg_out, sidx, didx, bufs, acc,
            gsems, ssems, zsem, deg2d, dega, iota_v):
    c = lax.axis_index("c")
    s = lax.axis_index("s")
    wid = c * NS + s
    row0 = s * RPT

    def _gather(chunk, b):
      pltpu.async_copy(y_hbm.at[sidx.at[chunk]], bufs[b], gsems[b])

    def _wait_gather(chunk, b):
      pltpu.make_async_copy(y_hbm.at[sidx.at[chunk]], bufs[b], gsems[b]).wait()

    def _scatter(chunk, b):
      pltpu.async_copy(bufs[b], acc.at[didx.at[chunk]], sem=ssems[b], add=True)

    def _wait_scatter(chunk, b):
      pltpu.make_async_copy(
          bufs[b], acc.at[didx.at[chunk]], ssems[b]).wait()

    pltpu.sync_copy(src_hbm.at[wid], sidx)
    pltpu.sync_copy(dst_hbm.at[wid], didx)
    _gather(0, 0)
    _gather(1, 1)
    inits = []

    def _init(nrows):
      pltpu.async_copy(zacc_hbm.at[pl.ds(row0, nrows)],
                       acc.at[pl.ds(row0, nrows)], zsem)

    pl.when(s < NS - 1)(lambda: _init(RPT))
    pl.when(s == NS - 1)(lambda: _init(RPT_LAST))
    if with_deg:
      drow = s * (DR // NS)
      inits.append(pltpu.async_copy(zdeg_hbm.at[pl.ds(0, DR)], deg2d, zsem))
      inits.append(pltpu.async_copy(zdeg_hbm.at[pl.ds(drow, DR // NS)],
                                    dega.at[pl.ds(drow, DR // NS)], zsem))
      inits.append(pltpu.async_copy(iota_hbm, iota_v, zsem))
    for d in inits:
      d.wait()
    pl.when(s < NS - 1)(
        lambda: pltpu.make_async_copy(zacc_hbm.at[pl.ds(row0, RPT)],
                                      acc.at[pl.ds(row0, RPT)], zsem).wait())
    pl.when(s == NS - 1)(
        lambda: pltpu.make_async_copy(zacc_hbm.at[pl.ds(row0, RPT_LAST)],
                                      acc.at[pl.ds(row0, RPT_LAST)],
                                      zsem).wait())
    plsc.subcore_barrier()

    @pl.loop(0, CPW - 1, step=4)
    def _loop(j):
      pl.when(j > 0)(lambda: _wait_scatter(j - 2, 2))
      _gather(j + 2, 2)
      pl.when(j > 0)(lambda: _wait_scatter(j - 1, 3))
      _gather(j + 3, 3)
      _wait_gather(j, 0)
      _scatter(j, 0)
      if with_deg:
        _deg_accum(deg2d, didx, j)
      _wait_gather(j + 1, 1)
      _scatter(j + 1, 1)
      if with_deg:
        _deg_accum(deg2d, didx, j + 1)
      _wait_scatter(j, 0)
      _gather(j + 4, 0)
      pl.when(j < CPW - 5)(lambda: (_wait_scatter(j + 1, 1),
                                    _gather(j + 5, 1), None)[-1])
      _wait_gather(j + 2, 2)
      _scatter(j + 2, 2)
      if with_deg:
        _deg_accum(deg2d, didx, j + 2)
      _wait_gather(j + 3, 3)
      _scatter(j + 3, 3)
      if with_deg:
        _deg_accum(deg2d, didx, j + 3)

    _wait_gather(CPW - 1, 0)
    _scatter(CPW - 1, 0)
    if with_deg:
      _deg_accum(deg2d, didx, CPW - 1)
    _wait_scatter(CPW - 4, 1)
    _wait_scatter(CPW - 3, 2)
    _wait_scatter(CPW - 2, 3)
    _wait_scatter(CPW - 1, 0)
    if with_deg:
      for b in range(DR // 128):
        pltpu.sync_copy(deg2d.at[pl.ds(b * 128, 128)],
                        dega.at[iota_v.at[b]], add=True)

    plsc.subcore_barrier()

    def _copy_out(nrows):
      if out_width == width:
        pltpu.sync_copy(acc.at[pl.ds(row0, nrows)],
                        acc_out.at[c, pl.ds(row0, nrows)])
      else:
        pltpu.sync_copy(acc.at[pl.ds(row0, nrows)],
                        acc_out.at[c, pl.ds(row0, nrows), pl.ds(0, width)])

    pl.when(s < NS - 1)(lambda: _copy_out(RPT))
    pl.when(s == NS - 1)(lambda: _copy_out(RPT_LAST))
    if with_deg:
      drow = s * (DR // NS)
      pltpu.sync_copy(dega.at[pl.ds(drow, DR // NS)],
                      deg_out.at[c, pl.ds(drow, DR // NS)])

  if with_deg:
    def fn(y_hbm, src_hbm, dst_hbm, zacc_hbm, zdeg_hbm, iota_hbm,
           acc_out, deg_out, sidx, didx, b0, b1, b2, b3, acc,
           g0, g1, g2, g3, s0, s1, s2, s3, zsem, deg2d, dega, iota_v):
      _body(y_hbm, src_hbm, dst_hbm, zacc_hbm, zdeg_hbm, iota_hbm,
            acc_out, deg_out, sidx, didx, (b0, b1, b2, b3), acc,
            (g0, g1, g2, g3), (s0, s1, s2, s3), zsem, deg2d, dega, iota_v)
    out = tuple(out_type)
  else:
    def fn(y_hbm, src_hbm, dst_hbm, zacc_hbm,
           acc_out, sidx, didx, b0, b1, b2, b3, acc,
           g0, g1, g2, g3, s0, s1, s2, s3, zsem):
      _body(y_hbm, src_hbm, dst_hbm, zacc_hbm, None, None,
            acc_out, None, sidx, didx, (b0, b1, b2, b3), acc,
            (g0, g1, g2, g3), (s0, s1, s2, s3), zsem, None, None, None)
    out = out_type[0]

  return pl.kernel(
      fn,
      out_type=out,
      mesh=mesh,
      scratch_types=scratch,
      compiler_params=pltpu.CompilerParams(
          use_tc_tiling_on_sc=False, needs_layout_passes=False),
  )


_SC_CACHE = {}


def _sc_aggregate(width, with_deg, out_width=None):
  key = (width, with_deg, out_width)
  if key not in _SC_CACHE:
    _SC_CACHE[key] = _make_sc_aggregate(width, with_deg, out_width)
  return _SC_CACHE[key]


def _tc1_body(x_ref, w_ref, b_ref, y_ref, r_ref):
  yr = jnp.dot(x_ref[...], w_ref[...], preferred_element_type=jnp.float32)
  y_ref[...] = yr[:, :H]
  r_ref[...] = yr[:, H:] + b_ref[...]


_tc1 = pl.pallas_call(
    _tc1_body,
    grid=(GRID,),
    in_specs=[
        pl.BlockSpec((BLK, D), lambda i: (i, 0)),
        pl.BlockSpec((D, 2 * H), lambda i: (0, 0)),
        pl.BlockSpec((1, H), lambda i: (0, 0)),
    ],
    out_specs=[
        pl.BlockSpec((BLK, H), lambda i: (i, 0)),
        pl.BlockSpec((BLK, H), lambda i: (i, 0)),
    ],
    out_shape=[
        jax.ShapeDtypeStruct((N, H), jnp.float32),
        jax.ShapeDtypeStruct((N, H), jnp.float32),
    ],
)


def _tc2_body(accp_ref, degp_ref, r1_ref, w_ref, b_ref, y2_ref, r2_ref):
  acc = accp_ref[0, :, :H] + accp_ref[1, :, :H]
  deg = degp_ref[0] + degp_ref[1]
  h = jnp.maximum(acc / jnp.maximum(deg, 1.0) + r1_ref[...], 0.0)
  yr = jnp.dot(h, w_ref[...], preferred_element_type=jnp.float32)
  y2_ref[...] = yr[:, :CP]
  r2_ref[...] = yr[:, CP:] + b_ref[...]


_tc2 = pl.pallas_call(
    _tc2_body,
    grid=(GRID,),
    in_specs=[
        pl.BlockSpec((NC, BLK, 2 * H), lambda i: (0, i, 0)),
        pl.BlockSpec((NC, BLK, 1), lambda i: (0, i, 0)),
        pl.BlockSpec((BLK, H), lambda i: (i, 0)),
        pl.BlockSpec((H, 2 * CP), lambda i: (0, 0)),
        pl.BlockSpec((1, CP), lambda i: (0, 0)),
    ],
    out_specs=[
        pl.BlockSpec((BLK, CP), lambda i: (i, 0)),
        pl.BlockSpec((BLK, CP), lambda i: (i, 0)),
    ],
    out_shape=[
        jax.ShapeDtypeStruct((N, CP), jnp.float32),
        jax.ShapeDtypeStruct((N, CP), jnp.float32),
    ],
)


def _tc3_body(accp_ref, degp_ref, r2_ref, out_ref):
  acc = accp_ref[0, :, :CP] + accp_ref[1, :, :CP]
  deg = degp_ref[0] + degp_ref[1]
  o = acc / jnp.maximum(deg, 1.0) + r2_ref[...]
  valid = lax.broadcasted_iota(jnp.int32, o.shape, 1) < 8
  om = jnp.where(valid, o, -jnp.inf)
  m = jnp.max(om, axis=1, keepdims=True)
  e = jnp.where(valid, jnp.exp(o - m), 0.0)
  lse = jnp.log(jnp.sum(e, axis=1, keepdims=True)) + m
  out_ref[...] = (o - lse)[:, :8]


_tc3 = pl.pallas_call(
    _tc3_body,
    grid=(GRID,),
    in_specs=[
        pl.BlockSpec((NC, BLK, 2 * H), lambda i: (0, i, 0)),
        pl.BlockSpec((NC, BLK, 1), lambda i: (0, i, 0)),
        pl.BlockSpec((BLK, CP), lambda i: (i, 0)),
    ],
    out_specs=pl.BlockSpec((BLK, 8), lambda i: (i, 0)),
    out_shape=jax.ShapeDtypeStruct((N, 8), jnp.float32),
)


def kernel(x, edge_index, Wl1, Wr1, b1, Wl2, Wr2, b2):
  src2d = edge_index[0].reshape(NW, CPW, CH)
  dst2d = edge_index[1].reshape(NW, CPW, CH)
  zH = jnp.zeros((N, H), jnp.float32)
  zC = jnp.zeros((N, CP), jnp.float32)
  iota = jnp.arange(DR, dtype=jnp.int32).reshape(DR // 128, 128)

  y1, r1 = _tc1(x, jnp.concatenate([Wl1, Wr1], axis=1), b1.reshape(1, H))
  accp, degg = _sc_aggregate(H, True, 2 * H)(y1, src2d, dst2d, zH, zC, iota)
  degp = degg.reshape(NC, DR * CP)[:, :N].reshape(NC, N, 1)

  W2 = jnp.concatenate(
      [jnp.pad(Wl2, ((0, 0), (0, CP - 8))),
       jnp.pad(Wr2, ((0, 0), (0, CP - 8)))], axis=1)
  b2p = jnp.pad(b2, (0, CP - 8)).reshape(1, CP)
  y2, r2 = _tc2(accp, degp, r1, W2, b2p)

  acc2p = _sc_aggregate(CP, False, 2 * H)(y2, src2d, dst2d, zC)
  return _tc3(acc2p, degp, r2)
